# baseline (device time: 31122 ns/iter reference)
import jax
import jax.numpy as jnp
from jax import lax
from jax.experimental import pallas as pl
from jax.experimental.pallas import tpu as pltpu

_N_CHUNKS = 8


def kernel(x, pi):
    _, m, n = x.shape
    rows = m // _N_CHUNKS

    def body(pi_ref, x_hbm, out_ref, stage, send_buf, local_sems, send_sems, recv_sems):
        my_x = lax.axis_index("x")
        my_y = lax.axis_index("y")
        my_z = lax.axis_index("z")
        target_z = pi_ref[my_z]
        swap = target_z != my_z

        def local_copy(k, slot):
            return pltpu.make_async_copy(
                x_hbm.at[0, pl.ds(k * rows, rows), :],
                stage.at[slot],
                local_sems.at[slot],
            )

        def store_out(k):
            return pltpu.make_async_copy(
                send_buf.at[pl.ds(k * rows, rows), :],
                out_ref.at[0, pl.ds(k * rows, rows), :],
                send_sems.at[k],
            )

        def rdma(k):
            return pltpu.make_async_remote_copy(
                src_ref=send_buf.at[pl.ds(k * rows, rows), :],
                dst_ref=out_ref.at[0, pl.ds(k * rows, rows), :],
                send_sem=send_sems.at[k],
                recv_sem=recv_sems.at[k],
                device_id=(my_x, my_y, target_z),
                device_id_type=pl.DeviceIdType.MESH,
            )

        barrier = pltpu.get_barrier_semaphore()
        pl.semaphore_signal(
            barrier,
            inc=1,
            device_id=(my_x, my_y, 1 - my_z),
            device_id_type=pl.DeviceIdType.MESH,
        )

        local_copy(0, 0).start()

        for k in range(_N_CHUNKS):
            slot = k % 2
            local_copy(k, slot).wait()
            if k + 1 < _N_CHUNKS:
                local_copy(k + 1, (k + 1) % 2).start()
            send_buf[pl.ds(k * rows, rows), :] = stage[slot].astype(jnp.bfloat16)

            if k == 0:
                pl.semaphore_wait(barrier, 1)

            @pl.when(swap)
            def _():
                rdma(k).start()

            @pl.when(jnp.logical_not(swap))
            def _():
                store_out(k).start()

        @pl.when(swap)
        def _():
            for k in range(_N_CHUNKS):
                d = rdma(k)
                d.wait_send()
                d.wait_recv()

        @pl.when(jnp.logical_not(swap))
        def _():
            for k in range(_N_CHUNKS):
                store_out(k).wait()

    return pl.pallas_call(
        body,
        out_shape=jax.ShapeDtypeStruct(x.shape, jnp.bfloat16),
        in_specs=[
            pl.BlockSpec(memory_space=pltpu.SMEM),
            pl.BlockSpec(memory_space=pltpu.MemorySpace.HBM),
        ],
        out_specs=pl.BlockSpec(memory_space=pltpu.MemorySpace.HBM),
        scratch_shapes=[
            pltpu.VMEM((2, rows, n), jnp.float32),
            pltpu.VMEM((m, n), jnp.bfloat16),
            pltpu.SemaphoreType.DMA((2,)),
            pltpu.SemaphoreType.DMA((_N_CHUNKS,)),
            pltpu.SemaphoreType.DMA((_N_CHUNKS,)),
        ],
        compiler_params=pltpu.CompilerParams(collective_id=0),
    )(pi, x)


# device time: 30995 ns/iter; 1.0041x vs baseline; 1.0041x over previous
import jax
import jax.numpy as jnp
from jax import lax
from jax.experimental import pallas as pl
from jax.experimental.pallas import tpu as pltpu

_N_CHUNKS = 8


def kernel(x, pi):
    _, m, n = x.shape
    rows = m // _N_CHUNKS

    def body(pi_ref, x_ref, out_ref, send_buf, send_sems, recv_sems):
        my_x = lax.axis_index("x")
        my_y = lax.axis_index("y")
        my_z = lax.axis_index("z")
        target_z = pi_ref[my_z]
        swap = target_z != my_z

        def store_out(k):
            return pltpu.make_async_copy(
                send_buf.at[pl.ds(k * rows, rows), :],
                out_ref.at[0, pl.ds(k * rows, rows), :],
                send_sems.at[k],
            )

        def rdma(k):
            return pltpu.make_async_remote_copy(
                src_ref=send_buf.at[pl.ds(k * rows, rows), :],
                dst_ref=out_ref.at[0, pl.ds(k * rows, rows), :],
                send_sem=send_sems.at[k],
                recv_sem=recv_sems.at[k],
                device_id=(my_x, my_y, target_z),
                device_id_type=pl.DeviceIdType.MESH,
            )

        barrier = pltpu.get_barrier_semaphore()
        pl.semaphore_signal(
            barrier,
            inc=1,
            device_id=(my_x, my_y, 1 - my_z),
            device_id_type=pl.DeviceIdType.MESH,
        )

        for k in range(_N_CHUNKS):
            sl = pl.ds(k * rows, rows)
            send_buf[sl, :] = x_ref[0, sl, :].astype(jnp.bfloat16)

            if k == 0:
                pl.semaphore_wait(barrier, 1)

            @pl.when(swap)
            def _():
                rdma(k).start()

            @pl.when(jnp.logical_not(swap))
            def _():
                store_out(k).start()

        @pl.when(swap)
        def _():
            for k in range(_N_CHUNKS):
                d = rdma(k)
                d.wait_send()
                d.wait_recv()

        @pl.when(jnp.logical_not(swap))
        def _():
            for k in range(_N_CHUNKS):
                store_out(k).wait()

    return pl.pallas_call(
        body,
        out_shape=jax.ShapeDtypeStruct(x.shape, jnp.bfloat16),
        in_specs=[
            pl.BlockSpec(memory_space=pltpu.SMEM),
            pl.BlockSpec(memory_space=pltpu.MemorySpace.VMEM),
        ],
        out_specs=pl.BlockSpec(memory_space=pltpu.MemorySpace.HBM),
        scratch_shapes=[
            pltpu.VMEM((m, n), jnp.bfloat16),
            pltpu.SemaphoreType.DMA((_N_CHUNKS,)),
            pltpu.SemaphoreType.DMA((_N_CHUNKS,)),
        ],
        compiler_params=pltpu.CompilerParams(collective_id=0),
    )(pi, x)
